# baseline (speedup 1.0000x reference)
import functools

import jax
import jax.numpy as jnp
from jax import lax
from jax.experimental import pallas as pl
from jax.experimental.pallas import tpu as pltpu
from jax.experimental.pallas import tpu_sc as plsc

B, L, V, D, H, C = 4096, 200, 1000000, 64, 256, 128

NC, NS = 2, 16
NW = NC * NS
RPW = B // NW
IPW = RPW * L
CI = 1024
NCH = IPW // CI

CB = 1536
SW = 128
KS = CB // SW

_mesh = plsc.VectorSubcoreMesh(core_axis_name="c", subcore_axis_name="s")


@functools.partial(
    pl.kernel,
    out_type=jax.ShapeDtypeStruct((B, D), jnp.float32),
    mesh=_mesh,
    compiler_params=pltpu.CompilerParams(use_tc_tiling_on_sc=False),
    scratch_types=[
        pltpu.VMEM((CI,), jnp.int32),
        pltpu.VMEM((CI,), jnp.int32),
        pltpu.VMEM((CI, D), jnp.float32),
        pltpu.VMEM_SHARED((NS * RPW, D), jnp.float32),
        pltpu.SemaphoreType.DMA,
    ],
)
def _sc_pool(xf, seg_hbm, zero_hbm, table, out, idx_v, seg_v, rows_v, acc_s,
             sem):
    sid = lax.axis_index("s")
    wid = sid * NC + lax.axis_index("c")
    pltpu.sync_copy(zero_hbm, acc_s.at[pl.ds(sid * RPW, RPW)])
    ibase = wid * IPW
    sbase = sid * IPW

    def body(i, carry):
        pltpu.sync_copy(xf.at[pl.ds(ibase + i * CI, CI)], idx_v)
        pltpu.sync_copy(seg_hbm.at[pl.ds(sbase + i * CI, CI)], seg_v)
        pltpu.async_copy(table.at[idx_v], rows_v, sem).wait()
        pltpu.sync_copy(rows_v, acc_s.at[seg_v], add=True)
        return carry

    lax.fori_loop(0, NCH, body, 0)
    pltpu.sync_copy(acc_s.at[pl.ds(sid * RPW, RPW)],
                    out.at[pl.ds(wid * RPW, RPW)])


def _tc_pack_body(t_ref, out_ref):
    jj = lax.broadcasted_iota(jnp.int32, (SW, SW), 0)
    ii = lax.broadcasted_iota(jnp.int32, (SW, SW), 1)
    p4 = (((jj < SW // 2) & (ii == 2 * jj))
          | ((jj >= SW // 2) & (ii == 2 * jj - (SW - 1)))).astype(jnp.float32)
    for s in range(0, KS, 2):
        ys2 = jnp.concatenate([t_ref[:, s * SW:(s + 1) * SW],
                               t_ref[:, (s + 1) * SW:(s + 2) * SW]], axis=0)
        r = lax.dot_general(p4, ys2, (((1,), (1,)), ((), ())),
                            preferred_element_type=jnp.float32)
        h = SW // 2
        out_ref[pl.ds(s * h, h), 0:D] = r[0:h, 0:D]
        out_ref[pl.ds(s * h, h), D:2 * D] = r[h:SW, 0:D]
        out_ref[pl.ds((s + 1) * h, h), 0:D] = r[0:h, D:2 * D]
        out_ref[pl.ds((s + 1) * h, h), D:2 * D] = r[h:SW, D:2 * D]


_tc_pack = pl.pallas_call(
    _tc_pack_body,
    grid=((V + CB - 1) // CB,),
    in_specs=[pl.BlockSpec((D, CB), lambda g: (0, g))],
    out_specs=pl.BlockSpec((CB // 2, 2 * D), lambda g: (g, 0)),
    out_shape=jax.ShapeDtypeStruct((V // 2, 2 * D), jnp.float32),
)


def _tc_head_body(msum_ref, W1_ref, b1_ref, gamma_ref, beta_ref, W2_ref,
                  b2_ref, out_ref):
    m = msum_ref[...] * (1.0 / L)
    h = lax.dot_general(m, W1_ref[...], (((1,), (1,)), ((), ())),
                        preferred_element_type=jnp.float32) + b1_ref[...]
    mu = jnp.mean(h, axis=0, keepdims=True)
    hc = h - mu
    var = jnp.mean(hc * hc, axis=0, keepdims=True)
    hn = hc * lax.rsqrt(var + 1e-5) * gamma_ref[...] + beta_ref[...]
    hr = jnp.maximum(hn, 0.0)
    out_ref[...] = lax.dot_general(hr, W2_ref[...], (((1,), (1,)), ((), ())),
                                   preferred_element_type=jnp.float32) + b2_ref[...]


_tc_head = pl.pallas_call(
    _tc_head_body,
    out_shape=jax.ShapeDtypeStruct((B, C), jnp.float32),
)


def kernel(x, table, W1, b1, gamma, beta, W2, b2):
    lt = _tc_pack(table.T).reshape(V, D)
    xf = x.astype(jnp.int32).reshape(B * L)
    seg = (lax.broadcasted_iota(jnp.int32, (NS, RPW, L), 1)
           + RPW * lax.broadcasted_iota(jnp.int32, (NS, RPW, L), 0)
           ).reshape(NS * IPW)
    zero = jnp.zeros((RPW, D), jnp.float32)
    msum = _sc_pool(xf, seg, zero, lt)
    return _tc_head(msum, W1, b1.reshape(1, H), gamma.reshape(1, H),
                    beta.reshape(1, H), W2, b2.reshape(1, C))


# speedup vs baseline: 1.1430x; 1.1430x over previous
import functools

import jax
import jax.numpy as jnp
from jax import lax
from jax.experimental import pallas as pl
from jax.experimental.pallas import tpu as pltpu
from jax.experimental.pallas import tpu_sc as plsc

B, L, V, D, H, C = 4096, 200, 1000000, 64, 256, 128

NC, NS = 2, 16
NW = NC * NS
RPW = B // NW
IPW = RPW * L
CI = 1024
NCH = IPW // CI

CB = 2048
SW = 128
KS = CB // SW

_mesh = plsc.VectorSubcoreMesh(core_axis_name="c", subcore_axis_name="s")


@functools.partial(
    pl.kernel,
    out_type=jax.ShapeDtypeStruct((B, D), jnp.float32),
    mesh=_mesh,
    compiler_params=pltpu.CompilerParams(use_tc_tiling_on_sc=False),
    scratch_types=[
        pltpu.VMEM((CI,), jnp.int32),
        pltpu.VMEM((CI,), jnp.int32),
        pltpu.VMEM((CI, D), jnp.float32),
        pltpu.VMEM_SHARED((NS * RPW, D), jnp.float32),
        pltpu.SemaphoreType.DMA,
    ],
)
def _sc_pool(xf, seg_hbm, zero_hbm, table, out, idx_v, seg_v, rows_v, acc_s,
             sem):
    sid = lax.axis_index("s")
    wid = sid * NC + lax.axis_index("c")
    pltpu.sync_copy(zero_hbm, acc_s.at[pl.ds(sid * RPW, RPW)])
    ibase = wid * IPW
    sbase = sid * IPW

    def body(i, carry):
        pltpu.sync_copy(xf.at[pl.ds(ibase + i * CI, CI)], idx_v)
        pltpu.sync_copy(seg_hbm.at[pl.ds(sbase + i * CI, CI)], seg_v)
        pltpu.async_copy(table.at[idx_v], rows_v, sem).wait()
        pltpu.sync_copy(rows_v, acc_s.at[seg_v], add=True)
        return carry

    lax.fori_loop(0, NCH, body, 0)
    pltpu.sync_copy(acc_s.at[pl.ds(sid * RPW, RPW)],
                    out.at[pl.ds(wid * RPW, RPW)])


def _tc_pack_body(t_ref, out_ref):
    jj = lax.broadcasted_iota(jnp.int32, (SW, SW), 0)
    ii = lax.broadcasted_iota(jnp.int32, (SW, SW), 1)
    p4 = (((jj < SW // 2) & (ii == 2 * jj))
          | ((jj >= SW // 2) & (ii == 2 * jj - (SW - 1)))).astype(jnp.float32)
    for s in range(KS):
        ys = t_ref[:, s * SW:(s + 1) * SW]
        r = lax.dot_general(p4, ys, (((1,), (1,)), ((), ())),
                            preferred_element_type=jnp.float32)
        out_ref[pl.ds(s * (SW // 2), SW // 2), 0:D] = r[0:SW // 2]
        out_ref[pl.ds(s * (SW // 2), SW // 2), D:2 * D] = r[SW // 2:SW]


_tc_pack = pl.pallas_call(
    _tc_pack_body,
    grid=((V + CB - 1) // CB,),
    in_specs=[pl.BlockSpec((D, CB), lambda g: (0, g))],
    out_specs=pl.BlockSpec((CB // 2, 2 * D), lambda g: (g, 0)),
    out_shape=jax.ShapeDtypeStruct((V // 2, 2 * D), jnp.float32),
)


def _tc_head_body(msum_ref, W1_ref, b1_ref, gamma_ref, beta_ref, W2_ref,
                  b2_ref, out_ref):
    m = msum_ref[...] * (1.0 / L)
    h = lax.dot_general(m, W1_ref[...], (((1,), (1,)), ((), ())),
                        preferred_element_type=jnp.float32) + b1_ref[...]
    mu = jnp.mean(h, axis=0, keepdims=True)
    hc = h - mu
    var = jnp.mean(hc * hc, axis=0, keepdims=True)
    hn = hc * lax.rsqrt(var + 1e-5) * gamma_ref[...] + beta_ref[...]
    hr = jnp.maximum(hn, 0.0)
    out_ref[...] = lax.dot_general(hr, W2_ref[...], (((1,), (1,)), ((), ())),
                                   preferred_element_type=jnp.float32) + b2_ref[...]


_tc_head = pl.pallas_call(
    _tc_head_body,
    out_shape=jax.ShapeDtypeStruct((B, C), jnp.float32),
)


def kernel(x, table, W1, b1, gamma, beta, W2, b2):
    lt = _tc_pack(table.T).reshape(V, D)
    xf = x.astype(jnp.int32).reshape(B * L)
    seg = (lax.broadcasted_iota(jnp.int32, (NS, RPW, L), 1)
           + RPW * lax.broadcasted_iota(jnp.int32, (NS, RPW, L), 0)
           ).reshape(NS * IPW)
    zero = jnp.zeros((RPW, D), jnp.float32)
    msum = _sc_pool(xf, seg, zero, lt)
    return _tc_head(msum, W1, b1.reshape(1, H), gamma.reshape(1, H),
                    beta.reshape(1, H), W2, b2.reshape(1, C))


# speedup vs baseline: 1.4266x; 1.2481x over previous
import functools

import jax
import jax.numpy as jnp
from jax import lax
from jax.experimental import pallas as pl
from jax.experimental.pallas import tpu as pltpu
from jax.experimental.pallas import tpu_sc as plsc

B, L, V, D, H, C = 4096, 200, 1000000, 64, 256, 128

NC, NS = 2, 16
NW = NC * NS
RPW = B // NW
IPW = RPW * L
CI = 1024
NCH = IPW // CI

CB = 4096
SW = 128
KS = CB // SW

_mesh = plsc.VectorSubcoreMesh(core_axis_name="c", subcore_axis_name="s")


@functools.partial(
    pl.kernel,
    out_type=jax.ShapeDtypeStruct((B, D), jnp.float32),
    mesh=_mesh,
    compiler_params=pltpu.CompilerParams(use_tc_tiling_on_sc=False),
    scratch_types=[
        pltpu.VMEM((CI,), jnp.int32),
        pltpu.VMEM((CI,), jnp.int32),
        pltpu.VMEM((CI, D), jnp.float32),
        pltpu.VMEM_SHARED((NS * RPW, D), jnp.float32),
        pltpu.SemaphoreType.DMA,
    ],
)
def _sc_pool(xf, seg_hbm, zero_hbm, table, out, idx_v, seg_v, rows_v, acc_s,
             sem):
    sid = lax.axis_index("s")
    wid = sid * NC + lax.axis_index("c")
    pltpu.sync_copy(zero_hbm, acc_s.at[pl.ds(sid * RPW, RPW)])
    ibase = wid * IPW
    sbase = sid * IPW

    def body(i, carry):
        pltpu.sync_copy(xf.at[pl.ds(ibase + i * CI, CI)], idx_v)
        pltpu.sync_copy(seg_hbm.at[pl.ds(sbase + i * CI, CI)], seg_v)
        pltpu.async_copy(table.at[idx_v], rows_v, sem).wait()
        pltpu.sync_copy(rows_v, acc_s.at[seg_v], add=True)
        return carry

    lax.fori_loop(0, NCH, body, 0)
    pltpu.sync_copy(acc_s.at[pl.ds(sid * RPW, RPW)],
                    out.at[pl.ds(wid * RPW, RPW)])


def _tc_pack_body(t_ref, out_ref):
    jj = lax.broadcasted_iota(jnp.int32, (SW, SW), 0)
    ii = lax.broadcasted_iota(jnp.int32, (SW, SW), 1)
    p4 = (((jj < SW // 2) & (ii == 2 * jj))
          | ((jj >= SW // 2) & (ii == 2 * jj - (SW - 1)))).astype(jnp.float32)
    for s in range(KS):
        ys = t_ref[:, s * SW:(s + 1) * SW]
        r = lax.dot_general(p4, ys, (((1,), (1,)), ((), ())),
                            preferred_element_type=jnp.float32)
        out_ref[pl.ds(s * (SW // 2), SW // 2), 0:D] = r[0:SW // 2]
        out_ref[pl.ds(s * (SW // 2), SW // 2), D:2 * D] = r[SW // 2:SW]


_tc_pack = pl.pallas_call(
    _tc_pack_body,
    grid=((V + CB - 1) // CB,),
    in_specs=[pl.BlockSpec((D, CB), lambda g: (0, g))],
    out_specs=pl.BlockSpec((CB // 2, 2 * D), lambda g: (g, 0)),
    out_shape=jax.ShapeDtypeStruct((V // 2, 2 * D), jnp.float32),
)


def _tc_head_body(msum_ref, W1_ref, b1_ref, gamma_ref, beta_ref, W2_ref,
                  b2_ref, out_ref):
    m = msum_ref[...] * (1.0 / L)
    h = lax.dot_general(m, W1_ref[...], (((1,), (1,)), ((), ())),
                        preferred_element_type=jnp.float32) + b1_ref[...]
    mu = jnp.mean(h, axis=0, keepdims=True)
    hc = h - mu
    var = jnp.mean(hc * hc, axis=0, keepdims=True)
    hn = hc * lax.rsqrt(var + 1e-5) * gamma_ref[...] + beta_ref[...]
    hr = jnp.maximum(hn, 0.0)
    out_ref[...] = lax.dot_general(hr, W2_ref[...], (((1,), (1,)), ((), ())),
                                   preferred_element_type=jnp.float32) + b2_ref[...]


_tc_head = pl.pallas_call(
    _tc_head_body,
    out_shape=jax.ShapeDtypeStruct((B, C), jnp.float32),
)


def kernel(x, table, W1, b1, gamma, beta, W2, b2):
    lt = _tc_pack(table.T).reshape(V, D)
    xf = x.astype(jnp.int32).reshape(B * L)
    seg = (lax.broadcasted_iota(jnp.int32, (NS, RPW, L), 1)
           + RPW * lax.broadcasted_iota(jnp.int32, (NS, RPW, L), 0)
           ).reshape(NS * IPW)
    zero = jnp.zeros((RPW, D), jnp.float32)
    msum = _sc_pool(xf, seg, zero, lt)
    return _tc_head(msum, W1, b1.reshape(1, H), gamma.reshape(1, H),
                    beta.reshape(1, H), W2, b2.reshape(1, C))


# speedup vs baseline: 1.6669x; 1.1685x over previous
import functools

import jax
import jax.numpy as jnp
from jax import lax
from jax.experimental import pallas as pl
from jax.experimental.pallas import tpu as pltpu
from jax.experimental.pallas import tpu_sc as plsc

B, L, V, D, H, C = 4096, 200, 1000000, 64, 256, 128

NC, NS = 2, 16
NW = NC * NS
RPW = B // NW
IPW = RPW * L
CI = 1024
NCH = IPW // CI

CB = 8192
SW = 128
KS = CB // SW

_mesh = plsc.VectorSubcoreMesh(core_axis_name="c", subcore_axis_name="s")


@functools.partial(
    pl.kernel,
    out_type=jax.ShapeDtypeStruct((B, D), jnp.float32),
    mesh=_mesh,
    compiler_params=pltpu.CompilerParams(use_tc_tiling_on_sc=False),
    scratch_types=[
        pltpu.VMEM((CI,), jnp.int32),
        pltpu.VMEM((CI,), jnp.int32),
        pltpu.VMEM((CI, D), jnp.float32),
        pltpu.VMEM_SHARED((NS * RPW, D), jnp.float32),
        pltpu.SemaphoreType.DMA,
    ],
)
def _sc_pool(xf, seg_hbm, zero_hbm, table, out, idx_v, seg_v, rows_v, acc_s,
             sem):
    sid = lax.axis_index("s")
    wid = sid * NC + lax.axis_index("c")
    pltpu.sync_copy(zero_hbm, acc_s.at[pl.ds(sid * RPW, RPW)])
    ibase = wid * IPW
    sbase = sid * IPW

    def body(i, carry):
        pltpu.sync_copy(xf.at[pl.ds(ibase + i * CI, CI)], idx_v)
        pltpu.sync_copy(seg_hbm.at[pl.ds(sbase + i * CI, CI)], seg_v)
        pltpu.async_copy(table.at[idx_v], rows_v, sem).wait()
        pltpu.sync_copy(rows_v, acc_s.at[seg_v], add=True)
        return carry

    lax.fori_loop(0, NCH, body, 0)
    pltpu.sync_copy(acc_s.at[pl.ds(sid * RPW, RPW)],
                    out.at[pl.ds(wid * RPW, RPW)])


def _tc_pack_body(t_ref, out_ref):
    jj = lax.broadcasted_iota(jnp.int32, (SW, SW), 0)
    ii = lax.broadcasted_iota(jnp.int32, (SW, SW), 1)
    p4 = (((jj < SW // 2) & (ii == 2 * jj))
          | ((jj >= SW // 2) & (ii == 2 * jj - (SW - 1)))).astype(jnp.float32)
    for s in range(KS):
        ys = t_ref[:, s * SW:(s + 1) * SW]
        r = lax.dot_general(p4, ys, (((1,), (1,)), ((), ())),
                            preferred_element_type=jnp.float32)
        out_ref[pl.ds(s * (SW // 2), SW // 2), 0:D] = r[0:SW // 2]
        out_ref[pl.ds(s * (SW // 2), SW // 2), D:2 * D] = r[SW // 2:SW]


_tc_pack = pl.pallas_call(
    _tc_pack_body,
    grid=((V + CB - 1) // CB,),
    in_specs=[pl.BlockSpec((D, CB), lambda g: (0, g))],
    out_specs=pl.BlockSpec((CB // 2, 2 * D), lambda g: (g, 0)),
    out_shape=jax.ShapeDtypeStruct((V // 2, 2 * D), jnp.float32),
)


def _tc_head_body(msum_ref, W1_ref, b1_ref, gamma_ref, beta_ref, W2_ref,
                  b2_ref, out_ref):
    m = msum_ref[...] * (1.0 / L)
    h = lax.dot_general(m, W1_ref[...], (((1,), (1,)), ((), ())),
                        preferred_element_type=jnp.float32) + b1_ref[...]
    mu = jnp.mean(h, axis=0, keepdims=True)
    hc = h - mu
    var = jnp.mean(hc * hc, axis=0, keepdims=True)
    hn = hc * lax.rsqrt(var + 1e-5) * gamma_ref[...] + beta_ref[...]
    hr = jnp.maximum(hn, 0.0)
    out_ref[...] = lax.dot_general(hr, W2_ref[...], (((1,), (1,)), ((), ())),
                                   preferred_element_type=jnp.float32) + b2_ref[...]


_tc_head = pl.pallas_call(
    _tc_head_body,
    out_shape=jax.ShapeDtypeStruct((B, C), jnp.float32),
)


def kernel(x, table, W1, b1, gamma, beta, W2, b2):
    lt = _tc_pack(table.T).reshape(V, D)
    xf = x.astype(jnp.int32).reshape(B * L)
    seg = (lax.broadcasted_iota(jnp.int32, (NS, RPW, L), 1)
           + RPW * lax.broadcasted_iota(jnp.int32, (NS, RPW, L), 0)
           ).reshape(NS * IPW)
    zero = jnp.zeros((RPW, D), jnp.float32)
    msum = _sc_pool(xf, seg, zero, lt)
    return _tc_head(msum, W1, b1.reshape(1, H), gamma.reshape(1, H),
                    beta.reshape(1, H), W2, b2.reshape(1, C))


# speedup vs baseline: 1.8108x; 1.0863x over previous
import functools

import jax
import jax.numpy as jnp
from jax import lax
from jax.experimental import pallas as pl
from jax.experimental.pallas import tpu as pltpu
from jax.experimental.pallas import tpu_sc as plsc

B, L, V, D, H, C = 4096, 200, 1000000, 64, 256, 128

NC, NS = 2, 16
NW = NC * NS
RPW = B // NW
IPW = RPW * L
CI = 1024
NCH = IPW // CI

CB = 16384
SW = 128
KS = CB // SW

_mesh = plsc.VectorSubcoreMesh(core_axis_name="c", subcore_axis_name="s")


@functools.partial(
    pl.kernel,
    out_type=jax.ShapeDtypeStruct((B, D), jnp.float32),
    mesh=_mesh,
    compiler_params=pltpu.CompilerParams(use_tc_tiling_on_sc=False),
    scratch_types=[
        pltpu.VMEM((CI,), jnp.int32),
        pltpu.VMEM((CI,), jnp.int32),
        pltpu.VMEM((CI, D), jnp.float32),
        pltpu.VMEM_SHARED((NS * RPW, D), jnp.float32),
        pltpu.SemaphoreType.DMA,
    ],
)
def _sc_pool(xf, seg_hbm, zero_hbm, table, out, idx_v, seg_v, rows_v, acc_s,
             sem):
    sid = lax.axis_index("s")
    wid = sid * NC + lax.axis_index("c")
    pltpu.sync_copy(zero_hbm, acc_s.at[pl.ds(sid * RPW, RPW)])
    ibase = wid * IPW
    sbase = sid * IPW

    def body(i, carry):
        pltpu.sync_copy(xf.at[pl.ds(ibase + i * CI, CI)], idx_v)
        pltpu.sync_copy(seg_hbm.at[pl.ds(sbase + i * CI, CI)], seg_v)
        pltpu.async_copy(table.at[idx_v], rows_v, sem).wait()
        pltpu.sync_copy(rows_v, acc_s.at[seg_v], add=True)
        return carry

    lax.fori_loop(0, NCH, body, 0)
    pltpu.sync_copy(acc_s.at[pl.ds(sid * RPW, RPW)],
                    out.at[pl.ds(wid * RPW, RPW)])


def _tc_pack_body(t_ref, out_ref):
    jj = lax.broadcasted_iota(jnp.int32, (SW, SW), 0)
    ii = lax.broadcasted_iota(jnp.int32, (SW, SW), 1)
    p4 = (((jj < SW // 2) & (ii == 2 * jj))
          | ((jj >= SW // 2) & (ii == 2 * jj - (SW - 1)))).astype(jnp.float32)
    for s in range(KS):
        ys = t_ref[:, s * SW:(s + 1) * SW]
        r = lax.dot_general(p4, ys, (((1,), (1,)), ((), ())),
                            preferred_element_type=jnp.float32)
        out_ref[pl.ds(s * (SW // 2), SW // 2), 0:D] = r[0:SW // 2]
        out_ref[pl.ds(s * (SW // 2), SW // 2), D:2 * D] = r[SW // 2:SW]


_tc_pack = pl.pallas_call(
    _tc_pack_body,
    grid=((V + CB - 1) // CB,),
    in_specs=[pl.BlockSpec((D, CB), lambda g: (0, g))],
    out_specs=pl.BlockSpec((CB // 2, 2 * D), lambda g: (g, 0)),
    out_shape=jax.ShapeDtypeStruct((V // 2, 2 * D), jnp.float32),
)


def _tc_head_body(msum_ref, W1_ref, b1_ref, gamma_ref, beta_ref, W2_ref,
                  b2_ref, out_ref):
    m = msum_ref[...] * (1.0 / L)
    h = lax.dot_general(m, W1_ref[...], (((1,), (1,)), ((), ())),
                        preferred_element_type=jnp.float32) + b1_ref[...]
    mu = jnp.mean(h, axis=0, keepdims=True)
    hc = h - mu
    var = jnp.mean(hc * hc, axis=0, keepdims=True)
    hn = hc * lax.rsqrt(var + 1e-5) * gamma_ref[...] + beta_ref[...]
    hr = jnp.maximum(hn, 0.0)
    out_ref[...] = lax.dot_general(hr, W2_ref[...], (((1,), (1,)), ((), ())),
                                   preferred_element_type=jnp.float32) + b2_ref[...]


_tc_head = pl.pallas_call(
    _tc_head_body,
    out_shape=jax.ShapeDtypeStruct((B, C), jnp.float32),
)


def kernel(x, table, W1, b1, gamma, beta, W2, b2):
    lt = _tc_pack(table.T).reshape(V, D)
    xf = x.astype(jnp.int32).reshape(B * L)
    seg = (lax.broadcasted_iota(jnp.int32, (NS, RPW, L), 1)
           + RPW * lax.broadcasted_iota(jnp.int32, (NS, RPW, L), 0)
           ).reshape(NS * IPW)
    zero = jnp.zeros((RPW, D), jnp.float32)
    msum = _sc_pool(xf, seg, zero, lt)
    return _tc_head(msum, W1, b1.reshape(1, H), gamma.reshape(1, H),
                    beta.reshape(1, H), W2, b2.reshape(1, C))


# speedup vs baseline: 1.8422x; 1.0173x over previous
import functools

import jax
import jax.numpy as jnp
from jax import lax
from jax.experimental import pallas as pl
from jax.experimental.pallas import tpu as pltpu
from jax.experimental.pallas import tpu_sc as plsc

B, L, V, D, H, C = 4096, 200, 1000000, 64, 256, 128

NC, NS = 2, 16
NW = NC * NS
RPW = B // NW
IPW = RPW * L
CI = 1024
NCH = IPW // CI

CB = 32768
SW = 128
KS = CB // SW

_mesh = plsc.VectorSubcoreMesh(core_axis_name="c", subcore_axis_name="s")


@functools.partial(
    pl.kernel,
    out_type=jax.ShapeDtypeStruct((B, D), jnp.float32),
    mesh=_mesh,
    compiler_params=pltpu.CompilerParams(use_tc_tiling_on_sc=False),
    scratch_types=[
        pltpu.VMEM((CI,), jnp.int32),
        pltpu.VMEM((CI,), jnp.int32),
        pltpu.VMEM((CI, D), jnp.float32),
        pltpu.VMEM_SHARED((NS * RPW, D), jnp.float32),
        pltpu.SemaphoreType.DMA,
    ],
)
def _sc_pool(xf, seg_hbm, zero_hbm, table, out, idx_v, seg_v, rows_v, acc_s,
             sem):
    sid = lax.axis_index("s")
    wid = sid * NC + lax.axis_index("c")
    pltpu.sync_copy(zero_hbm, acc_s.at[pl.ds(sid * RPW, RPW)])
    ibase = wid * IPW
    sbase = sid * IPW

    def body(i, carry):
        pltpu.sync_copy(xf.at[pl.ds(ibase + i * CI, CI)], idx_v)
        pltpu.sync_copy(seg_hbm.at[pl.ds(sbase + i * CI, CI)], seg_v)
        pltpu.async_copy(table.at[idx_v], rows_v, sem).wait()
        pltpu.sync_copy(rows_v, acc_s.at[seg_v], add=True)
        return carry

    lax.fori_loop(0, NCH, body, 0)
    pltpu.sync_copy(acc_s.at[pl.ds(sid * RPW, RPW)],
                    out.at[pl.ds(wid * RPW, RPW)])


def _tc_pack_body(t_ref, out_ref):
    jj = lax.broadcasted_iota(jnp.int32, (SW, SW), 0)
    ii = lax.broadcasted_iota(jnp.int32, (SW, SW), 1)
    p4 = (((jj < SW // 2) & (ii == 2 * jj))
          | ((jj >= SW // 2) & (ii == 2 * jj - (SW - 1)))).astype(jnp.float32)
    for s in range(KS):
        ys = t_ref[:, s * SW:(s + 1) * SW]
        r = lax.dot_general(p4, ys, (((1,), (1,)), ((), ())),
                            preferred_element_type=jnp.float32)
        out_ref[pl.ds(s * (SW // 2), SW // 2), 0:D] = r[0:SW // 2]
        out_ref[pl.ds(s * (SW // 2), SW // 2), D:2 * D] = r[SW // 2:SW]


_tc_pack = pl.pallas_call(
    _tc_pack_body,
    grid=((V + CB - 1) // CB,),
    in_specs=[pl.BlockSpec((D, CB), lambda g: (0, g))],
    out_specs=pl.BlockSpec((CB // 2, 2 * D), lambda g: (g, 0)),
    out_shape=jax.ShapeDtypeStruct((V // 2, 2 * D), jnp.float32),
)


def _tc_head_body(msum_ref, W1_ref, b1_ref, gamma_ref, beta_ref, W2_ref,
                  b2_ref, out_ref):
    m = msum_ref[...] * (1.0 / L)
    h = lax.dot_general(m, W1_ref[...], (((1,), (1,)), ((), ())),
                        preferred_element_type=jnp.float32) + b1_ref[...]
    mu = jnp.mean(h, axis=0, keepdims=True)
    hc = h - mu
    var = jnp.mean(hc * hc, axis=0, keepdims=True)
    hn = hc * lax.rsqrt(var + 1e-5) * gamma_ref[...] + beta_ref[...]
    hr = jnp.maximum(hn, 0.0)
    out_ref[...] = lax.dot_general(hr, W2_ref[...], (((1,), (1,)), ((), ())),
                                   preferred_element_type=jnp.float32) + b2_ref[...]


_tc_head = pl.pallas_call(
    _tc_head_body,
    out_shape=jax.ShapeDtypeStruct((B, C), jnp.float32),
)


def kernel(x, table, W1, b1, gamma, beta, W2, b2):
    lt = _tc_pack(table.T).reshape(V, D)
    xf = x.astype(jnp.int32).reshape(B * L)
    seg = (lax.broadcasted_iota(jnp.int32, (NS, RPW, L), 1)
           + RPW * lax.broadcasted_iota(jnp.int32, (NS, RPW, L), 0)
           ).reshape(NS * IPW)
    zero = jnp.zeros((RPW, D), jnp.float32)
    msum = _sc_pool(xf, seg, zero, lt)
    return _tc_head(msum, W1, b1.reshape(1, H), gamma.reshape(1, H),
                    beta.reshape(1, H), W2, b2.reshape(1, C))


# speedup vs baseline: 1.8692x; 1.0147x over previous
import functools

import jax
import jax.numpy as jnp
from jax import lax
from jax.experimental import pallas as pl
from jax.experimental.pallas import tpu as pltpu
from jax.experimental.pallas import tpu_sc as plsc

B, L, V, D, H, C = 4096, 200, 1000000, 64, 256, 128

NC, NS = 2, 16
NW = NC * NS
RPW = B // NW
IPW = RPW * L
CI = 1280
NCH = IPW // CI

CB = 32768
SW = 128
KS = CB // SW

_mesh = plsc.VectorSubcoreMesh(core_axis_name="c", subcore_axis_name="s")


@functools.partial(
    pl.kernel,
    out_type=jax.ShapeDtypeStruct((B, D), jnp.float32),
    mesh=_mesh,
    compiler_params=pltpu.CompilerParams(use_tc_tiling_on_sc=False),
    scratch_types=[
        pltpu.VMEM((CI,), jnp.int32),
        pltpu.VMEM((CI,), jnp.int32),
        pltpu.VMEM((CI, D), jnp.float32),
        pltpu.VMEM_SHARED((NS * RPW, D), jnp.float32),
        pltpu.SemaphoreType.DMA,
    ],
)
def _sc_pool(xf, seg_hbm, zero_hbm, table, out, idx_v, seg_v, rows_v, acc_s,
             sem):
    sid = lax.axis_index("s")
    wid = sid * NC + lax.axis_index("c")
    pltpu.sync_copy(zero_hbm, acc_s.at[pl.ds(sid * RPW, RPW)])
    ibase = wid * IPW
    sbase = sid * IPW

    def body(i, carry):
        pltpu.sync_copy(xf.at[pl.ds(ibase + i * CI, CI)], idx_v)
        pltpu.sync_copy(seg_hbm.at[pl.ds(sbase + i * CI, CI)], seg_v)
        pltpu.async_copy(table.at[idx_v], rows_v, sem).wait()
        pltpu.sync_copy(rows_v, acc_s.at[seg_v], add=True)
        return carry

    lax.fori_loop(0, NCH, body, 0)
    pltpu.sync_copy(acc_s.at[pl.ds(sid * RPW, RPW)],
                    out.at[pl.ds(wid * RPW, RPW)])


def _tc_pack_body(t_ref, out_ref):
    jj = lax.broadcasted_iota(jnp.int32, (SW, SW), 0)
    ii = lax.broadcasted_iota(jnp.int32, (SW, SW), 1)
    p4 = (((jj < SW // 2) & (ii == 2 * jj))
          | ((jj >= SW // 2) & (ii == 2 * jj - (SW - 1)))).astype(jnp.float32)
    for s in range(KS):
        ys = t_ref[:, s * SW:(s + 1) * SW]
        r = lax.dot_general(p4, ys, (((1,), (1,)), ((), ())),
                            preferred_element_type=jnp.float32)
        out_ref[pl.ds(s * (SW // 2), SW // 2), 0:D] = r[0:SW // 2]
        out_ref[pl.ds(s * (SW // 2), SW // 2), D:2 * D] = r[SW // 2:SW]


_tc_pack = pl.pallas_call(
    _tc_pack_body,
    grid=((V + CB - 1) // CB,),
    in_specs=[pl.BlockSpec((D, CB), lambda g: (0, g))],
    out_specs=pl.BlockSpec((CB // 2, 2 * D), lambda g: (g, 0)),
    out_shape=jax.ShapeDtypeStruct((V // 2, 2 * D), jnp.float32),
)


def _tc_head_body(msum_ref, W1_ref, b1_ref, gamma_ref, beta_ref, W2_ref,
                  b2_ref, out_ref):
    m = msum_ref[...] * (1.0 / L)
    h = lax.dot_general(m, W1_ref[...], (((1,), (1,)), ((), ())),
                        preferred_element_type=jnp.float32) + b1_ref[...]
    mu = jnp.mean(h, axis=0, keepdims=True)
    hc = h - mu
    var = jnp.mean(hc * hc, axis=0, keepdims=True)
    hn = hc * lax.rsqrt(var + 1e-5) * gamma_ref[...] + beta_ref[...]
    hr = jnp.maximum(hn, 0.0)
    out_ref[...] = lax.dot_general(hr, W2_ref[...], (((1,), (1,)), ((), ())),
                                   preferred_element_type=jnp.float32) + b2_ref[...]


_tc_head = pl.pallas_call(
    _tc_head_body,
    out_shape=jax.ShapeDtypeStruct((B, C), jnp.float32),
)


def kernel(x, table, W1, b1, gamma, beta, W2, b2):
    lt = _tc_pack(table.T).reshape(V, D)
    xf = x.astype(jnp.int32).reshape(B * L)
    seg = (lax.broadcasted_iota(jnp.int32, (NS, RPW, L), 1)
           + RPW * lax.broadcasted_iota(jnp.int32, (NS, RPW, L), 0)
           ).reshape(NS * IPW)
    zero = jnp.zeros((RPW, D), jnp.float32)
    msum = _sc_pool(xf, seg, zero, lt)
    return _tc_head(msum, W1, b1.reshape(1, H), gamma.reshape(1, H),
                    beta.reshape(1, H), W2, b2.reshape(1, C))


# speedup vs baseline: 1.8914x; 1.0119x over previous
import functools

import jax
import jax.numpy as jnp
from jax import lax
from jax.experimental import pallas as pl
from jax.experimental.pallas import tpu as pltpu
from jax.experimental.pallas import tpu_sc as plsc

B, L, V, D, H, C = 4096, 200, 1000000, 64, 256, 128

NC, NS = 2, 16
NW = NC * NS
RPW = B // NW
IPW = RPW * L
CI = 1600
NCH = IPW // CI

CB = 32768
SW = 128
KS = CB // SW

_mesh = plsc.VectorSubcoreMesh(core_axis_name="c", subcore_axis_name="s")


@functools.partial(
    pl.kernel,
    out_type=jax.ShapeDtypeStruct((B, D), jnp.float32),
    mesh=_mesh,
    compiler_params=pltpu.CompilerParams(use_tc_tiling_on_sc=False),
    scratch_types=[
        pltpu.VMEM((CI,), jnp.int32),
        pltpu.VMEM((CI,), jnp.int32),
        pltpu.VMEM((CI, D), jnp.float32),
        pltpu.VMEM_SHARED((NS * RPW, D), jnp.float32),
        pltpu.SemaphoreType.DMA,
    ],
)
def _sc_pool(xf, seg_hbm, zero_hbm, table, out, idx_v, seg_v, rows_v, acc_s,
             sem):
    sid = lax.axis_index("s")
    wid = sid * NC + lax.axis_index("c")
    pltpu.sync_copy(zero_hbm, acc_s.at[pl.ds(sid * RPW, RPW)])
    ibase = wid * IPW
    sbase = sid * IPW

    def body(i, carry):
        pltpu.sync_copy(xf.at[pl.ds(ibase + i * CI, CI)], idx_v)
        pltpu.sync_copy(seg_hbm.at[pl.ds(sbase + i * CI, CI)], seg_v)
        pltpu.async_copy(table.at[idx_v], rows_v, sem).wait()
        pltpu.sync_copy(rows_v, acc_s.at[seg_v], add=True)
        return carry

    lax.fori_loop(0, NCH, body, 0)
    pltpu.sync_copy(acc_s.at[pl.ds(sid * RPW, RPW)],
                    out.at[pl.ds(wid * RPW, RPW)])


def _tc_pack_body(t_ref, out_ref):
    jj = lax.broadcasted_iota(jnp.int32, (SW, SW), 0)
    ii = lax.broadcasted_iota(jnp.int32, (SW, SW), 1)
    p4 = (((jj < SW // 2) & (ii == 2 * jj))
          | ((jj >= SW // 2) & (ii == 2 * jj - (SW - 1)))).astype(jnp.float32)
    for s in range(KS):
        ys = t_ref[:, s * SW:(s + 1) * SW]
        r = lax.dot_general(p4, ys, (((1,), (1,)), ((), ())),
                            preferred_element_type=jnp.float32)
        out_ref[pl.ds(s * (SW // 2), SW // 2), 0:D] = r[0:SW // 2]
        out_ref[pl.ds(s * (SW // 2), SW // 2), D:2 * D] = r[SW // 2:SW]


_tc_pack = pl.pallas_call(
    _tc_pack_body,
    grid=((V + CB - 1) // CB,),
    in_specs=[pl.BlockSpec((D, CB), lambda g: (0, g))],
    out_specs=pl.BlockSpec((CB // 2, 2 * D), lambda g: (g, 0)),
    out_shape=jax.ShapeDtypeStruct((V // 2, 2 * D), jnp.float32),
)


def _tc_head_body(msum_ref, W1_ref, b1_ref, gamma_ref, beta_ref, W2_ref,
                  b2_ref, out_ref):
    m = msum_ref[...] * (1.0 / L)
    h = lax.dot_general(m, W1_ref[...], (((1,), (1,)), ((), ())),
                        preferred_element_type=jnp.float32) + b1_ref[...]
    mu = jnp.mean(h, axis=0, keepdims=True)
    hc = h - mu
    var = jnp.mean(hc * hc, axis=0, keepdims=True)
    hn = hc * lax.rsqrt(var + 1e-5) * gamma_ref[...] + beta_ref[...]
    hr = jnp.maximum(hn, 0.0)
    out_ref[...] = lax.dot_general(hr, W2_ref[...], (((1,), (1,)), ((), ())),
                                   preferred_element_type=jnp.float32) + b2_ref[...]


_tc_head = pl.pallas_call(
    _tc_head_body,
    out_shape=jax.ShapeDtypeStruct((B, C), jnp.float32),
)


def kernel(x, table, W1, b1, gamma, beta, W2, b2):
    lt = _tc_pack(table.T).reshape(V, D)
    xf = x.astype(jnp.int32).reshape(B * L)
    seg = (lax.broadcasted_iota(jnp.int32, (NS, RPW, L), 1)
           + RPW * lax.broadcasted_iota(jnp.int32, (NS, RPW, L), 0)
           ).reshape(NS * IPW)
    zero = jnp.zeros((RPW, D), jnp.float32)
    msum = _sc_pool(xf, seg, zero, lt)
    return _tc_head(msum, W1, b1.reshape(1, H), gamma.reshape(1, H),
                    beta.reshape(1, H), W2, b2.reshape(1, C))
